# baseline (speedup 1.0000x reference)
import functools

import jax
import jax.numpy as jnp
from jax import lax
from jax.experimental import pallas as pl
from jax.experimental.pallas import tpu as pltpu
from jax.experimental.pallas import tpu_sc as plsc

EDIM = 64
BATCH = 16384

_NC, _NS = 2, 16
_NW = _NC * _NS
_BPW = BATCH // _NW
_CH = 128
_NCH = _BPW // _CH

_mesh = plsc.VectorSubcoreMesh(core_axis_name="c", subcore_axis_name="s")


@functools.partial(
    pl.kernel,
    mesh=_mesh,
    out_type=jax.ShapeDtypeStruct((BATCH, EDIM), jnp.float32),
    scratch_types=[
        pltpu.VMEM((_BPW,), jnp.int32),
        pltpu.VMEM((_BPW, EDIM), jnp.float32),
        pltpu.SemaphoreType.DMA,
    ],
    compiler_params=pltpu.CompilerParams(use_tc_tiling_on_sc=False),
)
def _sc_gather(idx_hbm, table_hbm, out_hbm, idx_v, rows_v, sem):
    wid = lax.axis_index("s") * _NC + lax.axis_index("c")
    base = wid * _BPW
    pltpu.sync_copy(idx_hbm.at[pl.ds(base, _BPW)], idx_v)
    copies = [
        pltpu.async_copy(
            table_hbm.at[idx_v.at[pl.ds(j * _CH, _CH)]],
            rows_v.at[pl.ds(j * _CH, _CH)],
            sem,
        )
        for j in range(_NCH)
    ]
    for c in copies:
        c.wait()
    pltpu.sync_copy(rows_v, out_hbm.at[pl.ds(base, _BPW)])


_BLK = 2048


def _mlp_body(x_ref, w1_ref, b1_ref, w2_ref, b2_ref, o_ref):
    x = x_ref[...]
    h = jnp.dot(x, w1_ref[...], preferred_element_type=jnp.float32) + b1_ref[...]
    h = h * jax.nn.sigmoid(h)
    o_ref[...] = jnp.dot(h, w2_ref[...], preferred_element_type=jnp.float32) + b2_ref[...]


def _tc_mlp(emb, w1t, b1, w2t, b2):
    return pl.pallas_call(
        _mlp_body,
        grid=(BATCH // _BLK,),
        in_specs=[
            pl.BlockSpec((_BLK, EDIM), lambda i: (i, 0)),
            pl.BlockSpec((EDIM, EDIM), lambda i: (0, 0)),
            pl.BlockSpec((1, EDIM), lambda i: (0, 0)),
            pl.BlockSpec((EDIM, EDIM), lambda i: (0, 0)),
            pl.BlockSpec((1, EDIM), lambda i: (0, 0)),
        ],
        out_specs=pl.BlockSpec((_BLK, EDIM), lambda i: (i, 0)),
        out_shape=jax.ShapeDtypeStruct((BATCH, EDIM), jnp.float32),
    )(emb, w1t, b1.reshape(1, EDIM), w2t, b2.reshape(1, EDIM))


def kernel(index, table, W1, b1, W2, b2):
    emb = _sc_gather(index.astype(jnp.int32), table)
    return _tc_mlp(emb, W1.T, b1, W2.T, b2)


# speedup vs baseline: 2.9698x; 2.9698x over previous
import functools

import jax
import jax.numpy as jnp
from jax import lax
from jax.experimental import pallas as pl
from jax.experimental.pallas import tpu as pltpu
from jax.experimental.pallas import tpu_sc as plsc

EDIM = 64
BATCH = 16384

_NC, _NS = 2, 16
_NW = _NC * _NS
_BPW = BATCH // _NW
_L = 16
_R = 8
_NG = _BPW // _R

_mesh = plsc.VectorSubcoreMesh(core_axis_name="c", subcore_axis_name="s")


@functools.partial(
    pl.kernel,
    mesh=_mesh,
    out_type=jax.ShapeDtypeStruct((EDIM, BATCH), jnp.float32),
    scratch_types=[
        pltpu.VMEM((_BPW + _L,), jnp.int32),
        pltpu.VMEM((EDIM, _BPW), jnp.float32),
        [pltpu.VMEM((EDIM, 128), jnp.float32) for _ in range(_R)],
        [pltpu.SemaphoreType.DMA for _ in range(_R)],
    ],
    compiler_params=pltpu.CompilerParams(needs_layout_passes=False),
)
def _sc_gather_t(idx_hbm, tab_t_hbm, out_hbm, idx_v, cols_v, stages, sems):
    wid = lax.axis_index("s") * _NC + lax.axis_index("c")
    base = wid * _BPW
    pltpu.sync_copy(idx_hbm.at[pl.ds(base, _BPW)], idx_v.at[pl.ds(0, _BPW)])

    def fetch(ebase, lane, slot):
        v = idx_v[pl.ds(ebase, _L)]
        col = v[lane]
        tc = pl.multiple_of((col >> 7) << 7, 128)
        pltpu.make_async_copy(
            tab_t_hbm.at[:, pl.ds(tc, 128)], stages[slot], sems[slot]
        ).start()

    def drain(slot):
        pltpu.make_async_copy(
            tab_t_hbm.at[:, pl.ds(0, 128)], stages[slot], sems[slot]
        ).wait()

    def extract(g, slot):
        e = g * _R + slot
        v = idx_v[pl.ds(g * _R, _L)]
        c = jnp.full((_L,), v[slot] & 127, jnp.int32)
        p = jnp.full((_L,), e, jnp.int32)
        for k in range(EDIM // _L):
            d = lax.iota(jnp.int32, _L) + (k * _L)
            vals = plsc.load_gather(stages[slot], [d, c])
            plsc.store_scatter(cols_v, [d, p], vals)

    for slot in range(_R):
        fetch(0, slot, slot)

    def round_(g, carry):
        for slot in range(_R):
            drain(slot)
            extract(g, slot)

            @pl.when(g < _NG - 1)
            def _():
                fetch((g + 1) * _R, slot, slot)
        return carry

    lax.fori_loop(0, _NG, round_, 0)
    pltpu.sync_copy(cols_v, out_hbm.at[:, pl.ds(base, _BPW)])


_BLK = 2048


def _mlp_t_body(x_ref, w1_ref, b1_ref, w2_ref, b2_ref, o_ref):
    x = x_ref[...]
    h = jnp.dot(w1_ref[...], x, preferred_element_type=jnp.float32) + b1_ref[...]
    h = h * jax.nn.sigmoid(h)
    o_ref[...] = (
        jnp.dot(w2_ref[...], h, preferred_element_type=jnp.float32) + b2_ref[...]
    )


def _tc_mlp_t(emb_t, w1, b1c, w2, b2c):
    return pl.pallas_call(
        _mlp_t_body,
        grid=(BATCH // _BLK,),
        in_specs=[
            pl.BlockSpec((EDIM, _BLK), lambda i: (0, i)),
            pl.BlockSpec((EDIM, EDIM), lambda i: (0, 0)),
            pl.BlockSpec((EDIM, 1), lambda i: (0, 0)),
            pl.BlockSpec((EDIM, EDIM), lambda i: (0, 0)),
            pl.BlockSpec((EDIM, 1), lambda i: (0, 0)),
        ],
        out_specs=pl.BlockSpec((EDIM, _BLK), lambda i: (0, i)),
        out_shape=jax.ShapeDtypeStruct((EDIM, BATCH), jnp.float32),
    )(emb_t, w1, b1c, w2, b2c)


def kernel(index, table, W1, b1, W2, b2):
    emb_t = _sc_gather_t(index.astype(jnp.int32), table.T)
    out_t = _tc_mlp_t(emb_t, W1, b1[:, None], W2, b2[:, None])
    return out_t.T
